# baseline (speedup 1.0000x reference)
import functools

import jax
import jax.numpy as jnp
from jax import lax
from jax.experimental import pallas as pl
from jax.experimental.pallas import tpu as pltpu
from jax.experimental.pallas import tpu_sc as plsc

B = 4096
D = 1024
E = 8
K = 2
TM = 512
TM2 = 256
PMAX = K * B + E * TM2
NT2 = PMAX // TM2
NW = 32
TPW = B // NW
CT = 16


def _routing_body(x_ref, wg_ref, bg_ref, a0_ref, a1_ref, r0_ref, r1_ref,
                  w0_ref, w1_ref, cnt_ref, imp_ref, ll_ref, il_ref):
    i = pl.program_id(0)

    @pl.when(i == 0)
    def _():
        cnt_ref[...] = jnp.zeros((1, E), jnp.float32)
        imp_ref[...] = jnp.zeros((1, E), jnp.float32)

    cb = cnt_ref[...]
    x = x_ref[...]
    s = jax.lax.dot_general(
        x, wg_ref[...], (((1,), (1,)), ((), ())),
        preferred_element_type=jnp.float32) + bg_ref[...]
    ids = jax.lax.broadcasted_iota(jnp.int32, (TM, E), 1)
    m1 = jnp.max(s, axis=1, keepdims=True)
    a1v = jnp.min(jnp.where(s == m1, ids, E), axis=1, keepdims=True)
    s2 = jnp.where(ids == a1v, -jnp.inf, s)
    m2 = jnp.max(s2, axis=1, keepdims=True)
    a2v = jnp.min(jnp.where(s2 == m2, ids, E), axis=1, keepdims=True)
    e21 = jnp.exp(m2 - m1)
    ones16 = jnp.ones((TM, 16), jnp.float32)
    w0_ref[...] = (1.0 / (1.0 + e21)) * ones16
    w1_ref[...] = (e21 / (1.0 + e21)) * ones16
    a0_ref[...] = a1v
    a1_ref[...] = a2v
    is1 = (ids == a1v).astype(jnp.float32)
    is2 = (ids == a2v).astype(jnp.float32)
    m = is1 + is2
    c = m
    sh = 1
    while sh < TM:
        c = c + jnp.concatenate(
            [jnp.zeros((sh, E), jnp.float32), c[:TM - sh]], axis=0)
        sh *= 2
    cexc = c - m
    r0_ref[...] = jnp.sum(is1 * (cexc + cb), axis=1,
                          keepdims=True).astype(jnp.int32)
    r1_ref[...] = jnp.sum(is2 * (cexc + is1 + cb), axis=1,
                          keepdims=True).astype(jnp.int32)
    cnt_ref[...] = cb + jnp.sum(m, axis=0, keepdims=True)
    ex = jnp.exp(s - m1)
    sm = ex / jnp.sum(ex, axis=1, keepdims=True)
    imp_ref[...] += jnp.sum(sm, axis=0, keepdims=True)

    @pl.when(i == pl.num_programs(0) - 1)
    def _():
        cfin = cnt_ref[...]
        cm = jnp.sum(cfin) / E
        cvar = jnp.sum((cfin - cm) ** 2) / (E - 1)
        ll_ref[...] = cvar.reshape(1, 1) / (E * (B / E))
        im = imp_ref[...]
        imm = jnp.sum(im) / E
        ivar = jnp.sum((im - imm) ** 2) / (E - 1)
        il_ref[...] = ivar.reshape(1, 1) / (imm + 1e-8)


def _pos_body(a0_ref, a1_ref, r0_ref, r1_ref, cnt_ref,
              p0_ref, p1_ref, te_ref):
    c = cnt_ref[...]
    pc = jnp.ceil(c / TM2) * TM2
    lt = (jax.lax.broadcasted_iota(jnp.int32, (E, E), 0) <
          jax.lax.broadcasted_iota(jnp.int32, (E, E), 1)).astype(jnp.float32)
    offs = jax.lax.dot_general(pc, lt, (((1,), (0,)), ((), ())),
                               preferred_element_type=jnp.float32)
    iot = jax.lax.broadcasted_iota(jnp.int32, (TM, E), 1)
    for a_ref, r_ref, p_ref in ((a0_ref, r0_ref, p0_ref),
                                (a1_ref, r1_ref, p1_ref)):
        oh = (a_ref[...] == iot).astype(jnp.float32)
        osel = jnp.sum(oh * offs, axis=1, keepdims=True)
        p_ref[...] = osel.astype(jnp.int32) + r_ref[...]

    @pl.when(pl.program_id(0) == 0)
    def _():
        ends = offs + pc
        starts = (jax.lax.broadcasted_iota(jnp.int32, (NT2, 1), 0)
                  * TM2).astype(jnp.float32)
        cmp = (starts >= ends).astype(jnp.int32)
        te_ref[...] = jnp.minimum(jnp.sum(cmp, axis=1, keepdims=True), E - 1)


def _gmlp_body(te_ref, xs_ref, w1_ref, b1_ref, w2_ref, b2_ref, o_ref):
    xb = xs_ref[...]
    h = jax.lax.dot_general(
        xb, w1_ref[0], (((1,), (1,)), ((), ())),
        preferred_element_type=jnp.float32) + b1_ref[0]
    hb = jnp.maximum(h, 0.0).astype(jnp.bfloat16)
    o_ref[...] = jax.lax.dot_general(
        hb, w2_ref[0], (((1,), (1,)), ((), ())),
        preferred_element_type=jnp.float32) + b2_ref[0]


def _dispatch_body(xbf_hbm, p0_hbm, p1_hbm, xs_hbm,
                   rows_v, i0_v, i1_v, sem0, sem1):
    wid = lax.axis_index("s") * 2 + lax.axis_index("c")
    base = wid * TPW
    pltpu.sync_copy(p0_hbm.at[pl.ds(base, TPW)], i0_v)
    pltpu.sync_copy(p1_hbm.at[pl.ds(base, TPW)], i1_v)
    pltpu.sync_copy(xbf_hbm.at[pl.ds(base, TPW)], rows_v)
    c0 = pltpu.async_copy(rows_v, xs_hbm.at[i0_v], sem0)
    c1 = pltpu.async_copy(rows_v, xs_hbm.at[i1_v], sem1)
    c0.wait()
    c1.wait()


def _combine_body(os_hbm, p0_hbm, p1_hbm, w0_hbm, w1_hbm, out_hbm,
                  ia_v, ib_v, wa_v, wb_v, ra_v, rb_v, out_v, sema, semb):
    wid = lax.axis_index("s") * 2 + lax.axis_index("c")
    base = wid * TPW
    pltpu.sync_copy(w0_hbm.at[pl.ds(base, TPW)], wa_v)
    pltpu.sync_copy(w1_hbm.at[pl.ds(base, TPW)], wb_v)
    for ci in range(TPW // CT):
        tb = base + ci * CT
        pltpu.sync_copy(p0_hbm.at[pl.ds(tb, CT)], ia_v)
        pltpu.sync_copy(p1_hbm.at[pl.ds(tb, CT)], ib_v)
        ca = pltpu.async_copy(os_hbm.at[ia_v], ra_v, sema)
        cb = pltpu.async_copy(os_hbm.at[ib_v], rb_v, semb)
        ca.wait()
        cb.wait()

        def tok_body(t, carry):
            g0 = wa_v[ci * CT + t]
            g1 = wb_v[ci * CT + t]

            def d_body(dc, carry2):
                off = dc * 16
                a = ra_v[t, pl.ds(off, 16)]
                bvec = rb_v[t, pl.ds(off, 16)]
                out_v[t, pl.ds(off, 16)] = g0 * a + g1 * bvec
                return carry2

            return lax.fori_loop(0, D // 16, d_body, carry)

        lax.fori_loop(0, CT, tok_body, 0)
        pltpu.sync_copy(out_v, out_hbm.at[pl.ds(tb, CT)])


def kernel(x, Wg, bg, W1, b1, W2, b2):
    nt = B // TM
    f32 = jnp.float32
    a0, a1, r0, r1, w0, w1, cnt, imp, ll, il = pl.pallas_call(
        _routing_body,
        grid=(nt,),
        in_specs=[
            pl.BlockSpec((TM, D), lambda i: (i, 0)),
            pl.BlockSpec((E, D), lambda i: (0, 0)),
            pl.BlockSpec((1, E), lambda i: (0, 0)),
        ],
        out_specs=[pl.BlockSpec((TM, 1), lambda i: (i, 0))] * 4 + [
            pl.BlockSpec((TM, 16), lambda i: (i, 0)),
            pl.BlockSpec((TM, 16), lambda i: (i, 0)),
        ] + [
            pl.BlockSpec((1, E), lambda i: (0, 0)),
            pl.BlockSpec((1, E), lambda i: (0, 0)),
            pl.BlockSpec((1, 1), lambda i: (0, 0)),
            pl.BlockSpec((1, 1), lambda i: (0, 0)),
        ],
        out_shape=[
            jax.ShapeDtypeStruct((B, 1), jnp.int32),
            jax.ShapeDtypeStruct((B, 1), jnp.int32),
            jax.ShapeDtypeStruct((B, 1), jnp.int32),
            jax.ShapeDtypeStruct((B, 1), jnp.int32),
            jax.ShapeDtypeStruct((B, 16), f32),
            jax.ShapeDtypeStruct((B, 16), f32),
            jax.ShapeDtypeStruct((1, E), f32),
            jax.ShapeDtypeStruct((1, E), f32),
            jax.ShapeDtypeStruct((1, 1), f32),
            jax.ShapeDtypeStruct((1, 1), f32),
        ],
    )(x, Wg, bg.reshape(1, E))

    p0, p1, te = pl.pallas_call(
        _pos_body,
        grid=(nt,),
        in_specs=[pl.BlockSpec((TM, 1), lambda i: (i, 0))] * 4 + [
            pl.BlockSpec((1, E), lambda i: (0, 0)),
        ],
        out_specs=[
            pl.BlockSpec((TM, 1), lambda i: (i, 0)),
            pl.BlockSpec((TM, 1), lambda i: (i, 0)),
            pl.BlockSpec((NT2, 1), lambda i: (0, 0)),
        ],
        out_shape=[
            jax.ShapeDtypeStruct((B, 1), jnp.int32),
            jax.ShapeDtypeStruct((B, 1), jnp.int32),
            jax.ShapeDtypeStruct((NT2, 1), jnp.int32),
        ],
    )(a0, a1, r0, r1, cnt)

    p0f = p0.reshape(B)
    p1f = p1.reshape(B)
    x_bf = x.astype(jnp.bfloat16)
    x_i = jax.lax.bitcast_convert_type(x_bf.reshape(B, D // 2, 2), jnp.int32)
    xs_i = _sc_dispatch(x_i, p0f, p1f)
    xs = jax.lax.bitcast_convert_type(xs_i, jnp.bfloat16).reshape(PMAX, D)

    w1b = W1.astype(jnp.bfloat16)
    w2b = W2.astype(jnp.bfloat16)
    grid_spec = pltpu.PrefetchScalarGridSpec(
        num_scalar_prefetch=1,
        grid=(NT2,),
        in_specs=[
            pl.BlockSpec((TM2, D), lambda i, te_r: (i, 0)),
            pl.BlockSpec((1, D, D), lambda i, te_r: (te_r[i], 0, 0)),
            pl.BlockSpec((1, 1, D), lambda i, te_r: (te_r[i], 0, 0)),
            pl.BlockSpec((1, D, D), lambda i, te_r: (te_r[i], 0, 0)),
            pl.BlockSpec((1, 1, D), lambda i, te_r: (te_r[i], 0, 0)),
        ],
        out_specs=pl.BlockSpec((TM2, D), lambda i, te_r: (i, 0)),
    )
    os_rows = pl.pallas_call(
        _gmlp_body,
        grid_spec=grid_spec,
        out_shape=jax.ShapeDtypeStruct((PMAX, D), f32),
    )(te.reshape(NT2), xs, w1b, b1.reshape(E, 1, D), w2b, b2.reshape(E, 1, D))

    out = _sc_combine(os_rows, p0f, p1f, w0, w1)

    return out, ll.reshape(()), il.reshape(())


def _sc_mesh():
    return plsc.VectorSubcoreMesh(core_axis_name="c", subcore_axis_name="s",
                                  num_cores=2, num_subcores=16)


def _sc_dispatch(x_i, p0f, p1f):
    dispatch = functools.partial(
        pl.kernel,
        out_type=jax.ShapeDtypeStruct((PMAX, D // 2), jnp.int32),
        mesh=_sc_mesh(),
        scratch_types=[
            pltpu.VMEM((TPW, D // 2), jnp.int32),
            pltpu.VMEM((TPW,), jnp.int32),
            pltpu.VMEM((TPW,), jnp.int32),
            pltpu.SemaphoreType.DMA,
            pltpu.SemaphoreType.DMA,
        ],
    )(_dispatch_body)
    return dispatch(x_i, p0f, p1f)


def _sc_combine(os_rows, p0f, p1f, w0f, w1f):
    f32 = jnp.float32
    combine = functools.partial(
        pl.kernel,
        out_type=jax.ShapeDtypeStruct((B, D), f32),
        mesh=_sc_mesh(),
        scratch_types=[
            pltpu.VMEM((CT,), jnp.int32),
            pltpu.VMEM((CT,), jnp.int32),
            pltpu.VMEM((TPW, 16), f32),
            pltpu.VMEM((TPW, 16), f32),
            pltpu.VMEM((CT, D), f32),
            pltpu.VMEM((CT, D), f32),
            pltpu.VMEM((CT, D), f32),
            pltpu.SemaphoreType.DMA,
            pltpu.SemaphoreType.DMA,
        ],
    )(_combine_body)
    return combine(os_rows, p0f, p1f, w0f, w1f)


# speedup vs baseline: 2.3606x; 2.3606x over previous
import functools

import jax
import jax.numpy as jnp
from jax import lax
from jax.experimental import pallas as pl
from jax.experimental.pallas import tpu as pltpu
from jax.experimental.pallas import tpu_sc as plsc

B = 4096
D = 1024
E = 8
K = 2
TM = 512
TM2 = 256
PMAX = K * B + E * TM2
NT2 = PMAX // TM2
NW = 32
TPW = B // NW
CT = 16


def _routing_body(x_ref, wg_ref, bg_ref, a0_ref, a1_ref, r0_ref, r1_ref,
                  w0_ref, w1_ref, cnt_ref, imp_ref, ll_ref, il_ref):
    i = pl.program_id(0)

    @pl.when(i == 0)
    def _():
        cnt_ref[...] = jnp.zeros((1, E), jnp.float32)
        imp_ref[...] = jnp.zeros((1, E), jnp.float32)

    cb = cnt_ref[...]
    x = x_ref[...]
    s = jax.lax.dot_general(
        x, wg_ref[...], (((1,), (1,)), ((), ())),
        preferred_element_type=jnp.float32) + bg_ref[...]
    ids = jax.lax.broadcasted_iota(jnp.int32, (TM, E), 1)
    m1 = jnp.max(s, axis=1, keepdims=True)
    a1v = jnp.min(jnp.where(s == m1, ids, E), axis=1, keepdims=True)
    s2 = jnp.where(ids == a1v, -jnp.inf, s)
    m2 = jnp.max(s2, axis=1, keepdims=True)
    a2v = jnp.min(jnp.where(s2 == m2, ids, E), axis=1, keepdims=True)
    e21 = jnp.exp(m2 - m1)
    ones16 = jnp.ones((TM, 16), jnp.float32)
    w0_ref[...] = (1.0 / (1.0 + e21)) * ones16
    w1_ref[...] = (e21 / (1.0 + e21)) * ones16
    a0_ref[...] = a1v
    a1_ref[...] = a2v
    is1 = (ids == a1v).astype(jnp.float32)
    is2 = (ids == a2v).astype(jnp.float32)
    m = is1 + is2
    c = m
    sh = 1
    while sh < TM:
        c = c + jnp.concatenate(
            [jnp.zeros((sh, E), jnp.float32), c[:TM - sh]], axis=0)
        sh *= 2
    cexc = c - m
    r0_ref[...] = jnp.sum(is1 * (cexc + cb), axis=1,
                          keepdims=True).astype(jnp.int32)
    r1_ref[...] = jnp.sum(is2 * (cexc + is1 + cb), axis=1,
                          keepdims=True).astype(jnp.int32)
    cnt_ref[...] = cb + jnp.sum(m, axis=0, keepdims=True)
    ex = jnp.exp(s - m1)
    sm = ex / jnp.sum(ex, axis=1, keepdims=True)
    imp_ref[...] += jnp.sum(sm, axis=0, keepdims=True)

    @pl.when(i == pl.num_programs(0) - 1)
    def _():
        cfin = cnt_ref[...]
        cm = jnp.sum(cfin) / E
        cvar = jnp.sum((cfin - cm) ** 2) / (E - 1)
        ll_ref[...] = cvar.reshape(1, 1) / (E * (B / E))
        im = imp_ref[...]
        imm = jnp.sum(im) / E
        ivar = jnp.sum((im - imm) ** 2) / (E - 1)
        il_ref[...] = ivar.reshape(1, 1) / (imm + 1e-8)


def _pos_body(a0_ref, a1_ref, r0_ref, r1_ref, cnt_ref,
              p0_ref, p1_ref, te_ref):
    c = cnt_ref[...]
    pc = jnp.ceil(c / TM2) * TM2
    lt = (jax.lax.broadcasted_iota(jnp.int32, (E, E), 0) <
          jax.lax.broadcasted_iota(jnp.int32, (E, E), 1)).astype(jnp.float32)
    offs = jax.lax.dot_general(pc, lt, (((1,), (0,)), ((), ())),
                               preferred_element_type=jnp.float32)
    iot = jax.lax.broadcasted_iota(jnp.int32, (TM, E), 1)
    for a_ref, r_ref, p_ref in ((a0_ref, r0_ref, p0_ref),
                                (a1_ref, r1_ref, p1_ref)):
        oh = (a_ref[...] == iot).astype(jnp.float32)
        osel = jnp.sum(oh * offs, axis=1, keepdims=True)
        p_ref[...] = osel.astype(jnp.int32) + r_ref[...]

    @pl.when(pl.program_id(0) == 0)
    def _():
        ends = offs + pc
        starts = (jax.lax.broadcasted_iota(jnp.int32, (NT2, 1), 0)
                  * TM2).astype(jnp.float32)
        cmp = (starts >= ends).astype(jnp.int32)
        te_ref[...] = jnp.minimum(jnp.sum(cmp, axis=1, keepdims=True), E - 1)


def _gmlp_body(te_ref, xs_ref, w1_ref, b1_ref, w2_ref, b2_ref, o_ref):
    xb = xs_ref[...].astype(jnp.bfloat16)
    h = jax.lax.dot_general(
        xb, w1_ref[0], (((1,), (1,)), ((), ())),
        preferred_element_type=jnp.float32) + b1_ref[0]
    hb = jnp.maximum(h, 0.0).astype(jnp.bfloat16)
    o_ref[...] = jax.lax.dot_general(
        hb, w2_ref[0], (((1,), (1,)), ((), ())),
        preferred_element_type=jnp.float32) + b2_ref[0]


CH = 64


def _dispatch_body(x_hbm, p0_hbm, p1_hbm, xs_hbm,
                   rows_v, i0_v, i1_v, sem0, sem1):
    wid = lax.axis_index("s") * 2 + lax.axis_index("c")
    base = wid * TPW
    for ci in range(TPW // CH):
        cb = base + ci * CH
        pltpu.sync_copy(p0_hbm.at[pl.ds(cb, CH)], i0_v)
        pltpu.sync_copy(p1_hbm.at[pl.ds(cb, CH)], i1_v)
        pltpu.sync_copy(x_hbm.at[pl.ds(cb, CH)], rows_v)
        c0 = pltpu.async_copy(rows_v, xs_hbm.at[i0_v], sem0)
        c1 = pltpu.async_copy(rows_v, xs_hbm.at[i1_v], sem1)
        c0.wait()
        c1.wait()


def _combine_body(os_hbm, p0_hbm, p1_hbm, w0_hbm, w1_hbm, out_hbm,
                  ia_v, ib_v, wa_v, wb_v, ra_v, rb_v, out_v, sema, semb):
    wid = lax.axis_index("s") * 2 + lax.axis_index("c")
    base = wid * TPW
    pltpu.sync_copy(w0_hbm.at[pl.ds(base, TPW)], wa_v)
    pltpu.sync_copy(w1_hbm.at[pl.ds(base, TPW)], wb_v)
    for ci in range(TPW // CT):
        tb = base + ci * CT
        pltpu.sync_copy(p0_hbm.at[pl.ds(tb, CT)], ia_v)
        pltpu.sync_copy(p1_hbm.at[pl.ds(tb, CT)], ib_v)
        ca = pltpu.async_copy(os_hbm.at[ia_v], ra_v, sema)
        cb = pltpu.async_copy(os_hbm.at[ib_v], rb_v, semb)
        ca.wait()
        cb.wait()

        def tok_body(t, carry):
            g0 = wa_v[ci * CT + t]
            g1 = wb_v[ci * CT + t]
            for dc in range(D // 16):
                off = dc * 16
                a = ra_v[t, pl.ds(off, 16)]
                bvec = rb_v[t, pl.ds(off, 16)]
                out_v[t, pl.ds(off, 16)] = g0 * a + g1 * bvec
            return carry

        lax.fori_loop(0, CT, tok_body, 0)
        pltpu.sync_copy(out_v, out_hbm.at[pl.ds(tb, CT)])


def kernel(x, Wg, bg, W1, b1, W2, b2):
    nt = B // TM
    f32 = jnp.float32
    a0, a1, r0, r1, w0, w1, cnt, imp, ll, il = pl.pallas_call(
        _routing_body,
        grid=(nt,),
        in_specs=[
            pl.BlockSpec((TM, D), lambda i: (i, 0)),
            pl.BlockSpec((E, D), lambda i: (0, 0)),
            pl.BlockSpec((1, E), lambda i: (0, 0)),
        ],
        out_specs=[pl.BlockSpec((TM, 1), lambda i: (i, 0))] * 4 + [
            pl.BlockSpec((TM, 16), lambda i: (i, 0)),
            pl.BlockSpec((TM, 16), lambda i: (i, 0)),
        ] + [
            pl.BlockSpec((1, E), lambda i: (0, 0)),
            pl.BlockSpec((1, E), lambda i: (0, 0)),
            pl.BlockSpec((1, 1), lambda i: (0, 0)),
            pl.BlockSpec((1, 1), lambda i: (0, 0)),
        ],
        out_shape=[
            jax.ShapeDtypeStruct((B, 1), jnp.int32),
            jax.ShapeDtypeStruct((B, 1), jnp.int32),
            jax.ShapeDtypeStruct((B, 1), jnp.int32),
            jax.ShapeDtypeStruct((B, 1), jnp.int32),
            jax.ShapeDtypeStruct((B, 16), f32),
            jax.ShapeDtypeStruct((B, 16), f32),
            jax.ShapeDtypeStruct((1, E), f32),
            jax.ShapeDtypeStruct((1, E), f32),
            jax.ShapeDtypeStruct((1, 1), f32),
            jax.ShapeDtypeStruct((1, 1), f32),
        ],
    )(x, Wg, bg.reshape(1, E))

    p0, p1, te = pl.pallas_call(
        _pos_body,
        grid=(nt,),
        in_specs=[pl.BlockSpec((TM, 1), lambda i: (i, 0))] * 4 + [
            pl.BlockSpec((1, E), lambda i: (0, 0)),
        ],
        out_specs=[
            pl.BlockSpec((TM, 1), lambda i: (i, 0)),
            pl.BlockSpec((TM, 1), lambda i: (i, 0)),
            pl.BlockSpec((NT2, 1), lambda i: (0, 0)),
        ],
        out_shape=[
            jax.ShapeDtypeStruct((B, 1), jnp.int32),
            jax.ShapeDtypeStruct((B, 1), jnp.int32),
            jax.ShapeDtypeStruct((NT2, 1), jnp.int32),
        ],
    )(a0, a1, r0, r1, cnt)

    p0f = p0.reshape(B)
    p1f = p1.reshape(B)
    xs = _sc_dispatch(x, p0f, p1f)

    w1b = W1.astype(jnp.bfloat16)
    w2b = W2.astype(jnp.bfloat16)
    grid_spec = pltpu.PrefetchScalarGridSpec(
        num_scalar_prefetch=1,
        grid=(NT2,),
        in_specs=[
            pl.BlockSpec((TM2, D), lambda i, te_r: (i, 0)),
            pl.BlockSpec((1, D, D), lambda i, te_r: (te_r[i], 0, 0)),
            pl.BlockSpec((1, 1, D), lambda i, te_r: (te_r[i], 0, 0)),
            pl.BlockSpec((1, D, D), lambda i, te_r: (te_r[i], 0, 0)),
            pl.BlockSpec((1, 1, D), lambda i, te_r: (te_r[i], 0, 0)),
        ],
        out_specs=pl.BlockSpec((TM2, D), lambda i, te_r: (i, 0)),
    )
    os_rows = pl.pallas_call(
        _gmlp_body,
        grid_spec=grid_spec,
        out_shape=jax.ShapeDtypeStruct((PMAX, D), f32),
    )(te.reshape(NT2), xs, w1b, b1.reshape(E, 1, D), w2b, b2.reshape(E, 1, D))

    out = _sc_combine(os_rows, p0f, p1f, w0, w1)

    return out, ll.reshape(()), il.reshape(())


def _sc_mesh():
    return plsc.VectorSubcoreMesh(core_axis_name="c", subcore_axis_name="s",
                                  num_cores=2, num_subcores=16)


def _sc_dispatch(x, p0f, p1f):
    dispatch = functools.partial(
        pl.kernel,
        out_type=jax.ShapeDtypeStruct((PMAX, D), jnp.float32),
        mesh=_sc_mesh(),
        scratch_types=[
            pltpu.VMEM((CH, D), jnp.float32),
            pltpu.VMEM((CH,), jnp.int32),
            pltpu.VMEM((CH,), jnp.int32),
            pltpu.SemaphoreType.DMA,
            pltpu.SemaphoreType.DMA,
        ],
    )(_dispatch_body)
    return dispatch(x, p0f, p1f)


def _sc_combine(os_rows, p0f, p1f, w0f, w1f):
    f32 = jnp.float32
    combine = functools.partial(
        pl.kernel,
        out_type=jax.ShapeDtypeStruct((B, D), f32),
        mesh=_sc_mesh(),
        scratch_types=[
            pltpu.VMEM((CT,), jnp.int32),
            pltpu.VMEM((CT,), jnp.int32),
            pltpu.VMEM((TPW, 16), f32),
            pltpu.VMEM((TPW, 16), f32),
            pltpu.VMEM((CT, D), f32),
            pltpu.VMEM((CT, D), f32),
            pltpu.VMEM((CT, D), f32),
            pltpu.SemaphoreType.DMA,
            pltpu.SemaphoreType.DMA,
        ],
    )(_combine_body)
    return combine(os_rows, p0f, p1f, w0f, w1f)
